# baseline (speedup 1.0000x reference)
import functools

import jax
import jax.numpy as jnp
from jax import lax
from jax.experimental import pallas as pl
from jax.experimental.pallas import tpu as pltpu
from jax.experimental.pallas import tpu_sc as plsc

B, S, H, I, E, K, VOCAB = 1, 2048, 768, 1024, 8, 2, 100000
T = B * S
TB = T // 2
BV = 2048
NVB = (VOCAB + BV - 1) // BV

_bf16 = jnp.bfloat16
_f32 = jnp.float32


def _sc_gather(emb, ids):
    info = plsc.get_sparse_core_info()
    nc, ns = info.num_cores, info.num_subcores
    nw = nc * ns
    b_per_w = T // nw
    mesh = plsc.VectorSubcoreMesh(core_axis_name="c", subcore_axis_name="s")

    @functools.partial(
        pl.kernel,
        mesh=mesh,
        out_type=jax.ShapeDtypeStruct((T, H), _f32),
        scratch_types=[
            pltpu.VMEM((b_per_w,), jnp.int32),
            pltpu.VMEM((b_per_w, H), _f32),
            pltpu.SemaphoreType.DMA,
        ],
    )
    def k(emb_hbm, idx_hbm, out_hbm, idx_v, rows_v, sem):
        wid = lax.axis_index("s") * nc + lax.axis_index("c")
        base = wid * b_per_w
        pltpu.sync_copy(idx_hbm.at[pl.ds(base, b_per_w)], idx_v)
        pltpu.async_copy(emb_hbm.at[idx_v], rows_v, sem).wait()
        pltpu.sync_copy(rows_v, out_hbm.at[pl.ds(base, b_per_w)])

    return k(emb, ids)


def _router_body(hs_ref, rw_ref, rb_ref, coef_ref, aux_ref):
    x = hs_ref[...].astype(_bf16)
    w = rw_ref[...].astype(_bf16)
    rl = lax.dot_general(x, w, (((1,), (1,)), ((), ())),
                         preferred_element_type=_f32) + rb_ref[...]
    idx = lax.broadcasted_iota(jnp.int32, (T, E), 1)
    v1 = jnp.max(rl, axis=1, keepdims=True)
    i1 = jnp.min(jnp.where(rl == v1, idx, E), axis=1, keepdims=True)
    m1 = idx == i1
    rl2 = jnp.where(m1, -jnp.inf, rl)
    v2 = jnp.max(rl2, axis=1, keepdims=True)
    i2 = jnp.min(jnp.where(rl2 == v2, idx, E), axis=1, keepdims=True)
    m2 = idx == i2
    e = jnp.exp(v2 - v1)
    denom = 1.0 + e
    w1 = 1.0 / denom
    w2 = e / denom
    m1f = m1.astype(_f32)
    m2f = m2.astype(_f32)
    wm = m1f * w1 + m2f * w2
    cnt = jnp.sum(m1f + m2f, axis=0, keepdims=True)
    wsum = jnp.sum(wm, axis=0, keepdims=True)
    wsc = jnp.where(cnt > 0, wsum / jnp.maximum(cnt, 1.0), 0.0)
    coef_ref[...] = wsc * (m1f + m2f)
    aux = jnp.mean((cnt - T / E) ** 2) * 0.01
    aux_ref[...] = aux.reshape(1, 1)


def _router(hs, router_w, router_b):
    return pl.pallas_call(
        _router_body,
        out_shape=(
            jax.ShapeDtypeStruct((T, E), _f32),
            jax.ShapeDtypeStruct((1, 1), _f32),
        ),
    )(hs, router_w, router_b.reshape(1, E))


def _expert_body(hs_ref, coef_ref, w1_ref, b1_ref, w2_ref, b2_ref,
                 g_ref, b_ref, out_ref, acc_ref):
    i = pl.program_id(1)

    @pl.when(i == 0)
    def _():
        acc_ref[...] = jnp.zeros_like(acc_ref)

    x = hs_ref[...].astype(_bf16)
    h1 = lax.dot_general(x, w1_ref[0].astype(_bf16), (((1,), (1,)), ((), ())),
                         preferred_element_type=_f32) + b1_ref[0]
    h1 = h1 * (lax.erf(h1 / jnp.sqrt(_f32(2.0))) + 1.0) / 2.0
    h2 = lax.dot_general(h1.astype(_bf16), w2_ref[0].astype(_bf16),
                         (((1,), (1,)), ((), ())),
                         preferred_element_type=_f32) + b2_ref[0]
    eidx = lax.broadcasted_iota(jnp.int32, (TB, E), 1)
    ci = jnp.sum(coef_ref[...] * (eidx == i).astype(_f32), axis=1,
                 keepdims=True)
    acc_ref[...] += h2 * ci

    @pl.when(i == E - 1)
    def _():
        o = acc_ref[...]
        mu = jnp.mean(o, axis=1, keepdims=True)
        var = jnp.mean((o - mu) ** 2, axis=1, keepdims=True)
        ln = (o - mu) / jnp.sqrt(var + 1e-5) * g_ref[...] + b_ref[...]
        out_ref[...] = ln.astype(_bf16)


def _experts(hs, coef, fc1_w, fc1_b, fc2_w, fc2_b, ln_g, ln_b):
    return pl.pallas_call(
        _expert_body,
        grid=(T // TB, E),
        in_specs=[
            pl.BlockSpec((TB, H), lambda t, i: (t, 0)),
            pl.BlockSpec((TB, E), lambda t, i: (t, 0)),
            pl.BlockSpec((1, I, H), lambda t, i: (i, 0, 0)),
            pl.BlockSpec((1, 1, I), lambda t, i: (i, 0, 0)),
            pl.BlockSpec((1, H, I), lambda t, i: (i, 0, 0)),
            pl.BlockSpec((1, 1, H), lambda t, i: (i, 0, 0)),
            pl.BlockSpec((1, H), lambda t, i: (0, 0)),
            pl.BlockSpec((1, H), lambda t, i: (0, 0)),
        ],
        out_specs=pl.BlockSpec((TB, H), lambda t, i: (t, 0)),
        out_shape=jax.ShapeDtypeStruct((T, H), _bf16),
        scratch_shapes=[pltpu.VMEM((TB, H), _f32)],
        compiler_params=pltpu.CompilerParams(
            dimension_semantics=("parallel", "arbitrary")),
    )(hs, coef, fc1_w, fc1_b.reshape(E, 1, I), fc2_w, fc2_b.reshape(E, 1, H),
      ln_g.reshape(1, H), ln_b.reshape(1, H))


def _head_body(ln_ref, w_ref, b_ref, out_ref):
    out_ref[...] = lax.dot_general(
        ln_ref[...], w_ref[...].astype(_bf16), (((1,), (1,)), ((), ())),
        preferred_element_type=_f32) + b_ref[...]


def _head(ln, head_w, head_b):
    return pl.pallas_call(
        _head_body,
        grid=(NVB,),
        in_specs=[
            pl.BlockSpec((T, H), lambda v: (0, 0)),
            pl.BlockSpec((BV, H), lambda v: (v, 0)),
            pl.BlockSpec((1, BV), lambda v: (0, v)),
        ],
        out_specs=pl.BlockSpec((T, BV), lambda v: (0, v)),
        out_shape=jax.ShapeDtypeStruct((T, VOCAB), _f32),
        compiler_params=pltpu.CompilerParams(
            dimension_semantics=("parallel",)),
    )(ln, head_w, head_b.reshape(1, VOCAB))


def kernel(input_ids, emb, router_w, router_b, fc1_w, fc1_b, fc2_w, fc2_b,
           ln_g, ln_b, head_w, head_b):
    ids = input_ids.reshape(-1).astype(jnp.int32)
    hs = _sc_gather(emb, ids)
    coef, aux = _router(hs, router_w, router_b)
    ln = _experts(hs, coef, fc1_w, fc1_b, fc2_w, fc2_b, ln_g, ln_b)
    logits = _head(ln, head_w, head_b)
    return logits.reshape(B, S, VOCAB), aux.reshape(())
